# baseline (speedup 1.0000x reference)
import jax
import jax.numpy as jnp
from jax import lax
from jax.experimental import pallas as pl
from jax.experimental.pallas import tpu as pltpu
from jax.experimental.pallas import tpu_sc as plsc

N_NODES = 10000
N_EDGES = 320000
D_FEAT = 128
DH = D_FEAT // 2
NS = 16
NC = 2
L = 16
CHUNK = 128
CHUNKS_PER_TILE = -(-N_EDGES // (NS * CHUNK))
EDGES_PER_TILE = CHUNKS_PER_TILE * CHUNK
E_PAD = EDGES_PER_TILE * NS
ROWS_PER_TILE = N_NODES // NS
OUT_CHUNK = 25


NBUF = 3


def _sc_body(tbl, w3, gi3, si3, out, sidx_v, gidx_v, w_v, rows4, tmp_v, accum,
             gsem, ssem):
    c = lax.axis_index("c")
    s = lax.axis_index("s")

    pltpu.sync_copy(si3.at[s], sidx_v)
    pltpu.sync_copy(gi3.at[s], gidx_v)
    pltpu.sync_copy(w3.at[s], w_v)

    def _fix_row(g, carry):
        for k in range(CHUNK // L):
            sl = pl.ds(k * L, L)
            gidx_v[g, sl] = gidx_v[g, sl] * 2 + c
        return carry
    lax.fori_loop(0, CHUNKS_PER_TILE, _fix_row, 0)

    zv = jnp.zeros((L,), jnp.float32)
    def _zero_row(r, carry):
        for k in range(DH // L):
            tmp_v[r, pl.ds(k * L, L)] = zv
        return carry
    lax.fori_loop(0, OUT_CHUNK, _zero_row, 0)
    base = s * ROWS_PER_TILE
    for i in range(ROWS_PER_TILE // OUT_CHUNK):
        pltpu.sync_copy(tmp_v, accum.at[pl.ds(base + i * OUT_CHUNK, OUT_CHUNK)])
    plsc.subcore_barrier()

    def _issue_gather(g, b):
        pltpu.async_copy(tbl.at[gidx_v.at[g]], rows4.at[b], gsem.at[b])

    def _wait_gather(g, b):
        pltpu.make_async_copy(tbl.at[gidx_v.at[g]], rows4.at[b],
                              gsem.at[b]).wait()

    def _issue_scatter(g, b):
        pltpu.async_copy(rows4.at[b], accum.at[sidx_v.at[g]], ssem.at[b],
                         add=True)

    def _wait_scatter(g, b):
        pltpu.make_async_copy(rows4.at[b], accum.at[sidx_v.at[g]],
                              ssem.at[b]).wait()

    _issue_gather(0, 0)
    _issue_gather(1, 1)

    def _chunk(g, carry):
        b = lax.rem(g, NBUF)
        bn = lax.rem(g + 2, NBUF)

        _wait_gather(g, b)

        def _group(q, carry2):
            wv = w_v[g, pl.ds(q * L, L)]
            j0 = q * L
            for jj in range(L):
                wj = wv[jj]
                for k in range(DH // L):
                    sl = pl.ds(k * L, L)
                    rows4[b, j0 + jj, sl] = rows4[b, j0 + jj, sl] * wj
            return carry2
        lax.fori_loop(0, CHUNK // L, _group, 0)

        _issue_scatter(g, b)

        @pl.when(g >= 1)
        def _():
            _wait_scatter(g - 1, bn)

        @pl.when(g + 2 < CHUNKS_PER_TILE)
        def _():
            _issue_gather(g + 2, bn)

        return carry
    lax.fori_loop(0, CHUNKS_PER_TILE, _chunk, 0)
    _wait_scatter(CHUNKS_PER_TILE - 1, (CHUNKS_PER_TILE - 1) % NBUF)
    plsc.subcore_barrier()

    for i in range(ROWS_PER_TILE // OUT_CHUNK):
        r0 = base + i * OUT_CHUNK
        pltpu.sync_copy(accum.at[pl.ds(r0, OUT_CHUNK)], tmp_v)
        def _relu_row(r, carry):
            for k in range(DH // L):
                sl = pl.ds(k * L, L)
                tmp_v[r, sl] = jnp.maximum(tmp_v[r, sl], 0.0)
            return carry
        lax.fori_loop(0, OUT_CHUNK, _relu_row, 0)
        pltpu.sync_copy(tmp_v, out.at[pl.ds(r0, OUT_CHUNK), c])


def kernel(nodes, edges, senders, receivers):
    tbl = nodes.reshape(N_NODES * 2, DH)
    pad = E_PAD - N_EDGES
    w3 = jnp.pad(edges.reshape(N_EDGES), (0, pad)).reshape(
        NS, CHUNKS_PER_TILE, CHUNK)
    si3 = jnp.pad(senders, (0, pad)).reshape(NS, CHUNKS_PER_TILE, CHUNK)
    gi3 = jnp.pad(receivers, (0, pad)).reshape(NS, CHUNKS_PER_TILE, CHUNK)

    mesh = plsc.VectorSubcoreMesh(
        core_axis_name="c", subcore_axis_name="s",
        num_cores=NC, num_subcores=NS)
    out = pl.kernel(
        _sc_body,
        out_type=jax.ShapeDtypeStruct((N_NODES, NC, DH), jnp.float32),
        mesh=mesh,
        compiler_params=pltpu.CompilerParams(use_tc_tiling_on_sc=False),
        scratch_types=[
            pltpu.VMEM((CHUNKS_PER_TILE, CHUNK), jnp.int32),
            pltpu.VMEM((CHUNKS_PER_TILE, CHUNK), jnp.int32),
            pltpu.VMEM((CHUNKS_PER_TILE, CHUNK), jnp.float32),
            pltpu.VMEM((NBUF, CHUNK, DH), jnp.float32),
            pltpu.VMEM((OUT_CHUNK, DH), jnp.float32),
            pltpu.VMEM_SHARED((N_NODES, DH), jnp.float32),
            pltpu.SemaphoreType.DMA((NBUF,)),
            pltpu.SemaphoreType.DMA((NBUF,)),
        ],
    )(tbl, w3, gi3, si3)
    return out.reshape(N_NODES, D_FEAT)


# speedup vs baseline: 1.2445x; 1.2445x over previous
import jax
import jax.numpy as jnp
from jax import lax
from jax.experimental import pallas as pl
from jax.experimental.pallas import tpu as pltpu
from jax.experimental.pallas import tpu_sc as plsc

N_NODES = 10000
N_EDGES = 320000
D_FEAT = 128
DH = D_FEAT // 2
NS = 16
NC = 2
L = 16
CHUNK = 128
N_CHUNKS = N_EDGES // CHUNK
CHUNKS_PER_TILE = -(-N_CHUNKS // NS)
REM = N_CHUNKS - (N_CHUNKS // NS) * NS
ROWS_PER_TILE = N_NODES // NS
OUT_CHUNK = 25


NBUF = 3


def _sc_body(tbl, w3, gi3, si3, out, sidx_v, gidx_v, w_v, rows4, tmp_v, accum,
             gsem, ssem):
    c = lax.axis_index("c")
    s = lax.axis_index("s")

    n_t = (N_CHUNKS // NS) + jnp.where(s < REM, 1, 0)
    start = s * (N_CHUNKS // NS) + jnp.minimum(s, REM)
    stage_base = jnp.minimum(start, N_CHUNKS - CHUNKS_PER_TILE)
    lo = start - stage_base
    hi = lo + n_t

    pltpu.sync_copy(si3.at[pl.ds(stage_base, CHUNKS_PER_TILE)], sidx_v)
    pltpu.sync_copy(gi3.at[pl.ds(stage_base, CHUNKS_PER_TILE)], gidx_v)
    pltpu.sync_copy(w3.at[pl.ds(stage_base, CHUNKS_PER_TILE)], w_v)

    def _fix_row(g, carry):
        for k in range(CHUNK // L):
            sl = pl.ds(k * L, L)
            gidx_v[g, sl] = gidx_v[g, sl] * 2 + c
        return carry
    lax.fori_loop(0, CHUNKS_PER_TILE, _fix_row, 0)

    zv = jnp.zeros((L,), jnp.float32)
    def _zero_row(r, carry):
        for k in range(DH // L):
            tmp_v[r, pl.ds(k * L, L)] = zv
        return carry
    lax.fori_loop(0, OUT_CHUNK, _zero_row, 0)
    base = s * ROWS_PER_TILE
    for i in range(ROWS_PER_TILE // OUT_CHUNK):
        pltpu.sync_copy(tmp_v, accum.at[pl.ds(base + i * OUT_CHUNK, OUT_CHUNK)])
    plsc.subcore_barrier()

    def _issue_gather(g, b):
        pltpu.async_copy(tbl.at[gidx_v.at[g]], rows4.at[b], gsem.at[b])

    def _wait_gather(g, b):
        pltpu.make_async_copy(tbl.at[gidx_v.at[g]], rows4.at[b],
                              gsem.at[b]).wait()

    def _issue_scatter(g, b):
        pltpu.async_copy(rows4.at[b], accum.at[sidx_v.at[g]], ssem.at[b],
                         add=True)

    def _wait_scatter(g, b):
        pltpu.make_async_copy(rows4.at[b], accum.at[sidx_v.at[g]],
                              ssem.at[b]).wait()

    _issue_gather(lo, lax.rem(lo, NBUF))
    _issue_gather(lo + 1, lax.rem(lo + 1, NBUF))

    def _chunk(g, carry):
        b = lax.rem(g, NBUF)
        bn = lax.rem(g + 2, NBUF)

        _wait_gather(g, b)

        def _group(q, carry2):
            wv = w_v[g, pl.ds(q * L, L)]
            j0 = q * L
            for jj in range(L):
                wj = wv[jj]
                for k in range(DH // L):
                    sl = pl.ds(k * L, L)
                    rows4[b, j0 + jj, sl] = rows4[b, j0 + jj, sl] * wj
            return carry2
        lax.fori_loop(0, CHUNK // L, _group, 0)

        _issue_scatter(g, b)

        @pl.when(g >= lo + 1)
        def _():
            _wait_scatter(g - 1, bn)

        @pl.when(g + 2 < hi)
        def _():
            _issue_gather(g + 2, bn)

        return carry
    lax.fori_loop(lo, hi, _chunk, 0)
    _wait_scatter(hi - 1, lax.rem(hi - 1, NBUF))
    plsc.subcore_barrier()

    for i in range(ROWS_PER_TILE // OUT_CHUNK):
        r0 = base + i * OUT_CHUNK
        pltpu.sync_copy(accum.at[pl.ds(r0, OUT_CHUNK)], tmp_v)
        def _relu_row(r, carry):
            for k in range(DH // L):
                sl = pl.ds(k * L, L)
                tmp_v[r, sl] = jnp.maximum(tmp_v[r, sl], 0.0)
            return carry
        lax.fori_loop(0, OUT_CHUNK, _relu_row, 0)
        pltpu.sync_copy(tmp_v, out.at[pl.ds(r0, OUT_CHUNK), c])


def kernel(nodes, edges, senders, receivers):
    tbl = nodes.reshape(N_NODES * 2, DH)
    w3 = edges.reshape(N_CHUNKS, CHUNK)
    si3 = senders.reshape(N_CHUNKS, CHUNK)
    gi3 = receivers.reshape(N_CHUNKS, CHUNK)

    mesh = plsc.VectorSubcoreMesh(
        core_axis_name="c", subcore_axis_name="s",
        num_cores=NC, num_subcores=NS)
    out = pl.kernel(
        _sc_body,
        out_type=jax.ShapeDtypeStruct((N_NODES, NC, DH), jnp.float32),
        mesh=mesh,
        compiler_params=pltpu.CompilerParams(use_tc_tiling_on_sc=False),
        scratch_types=[
            pltpu.VMEM((CHUNKS_PER_TILE, CHUNK), jnp.int32),
            pltpu.VMEM((CHUNKS_PER_TILE, CHUNK), jnp.int32),
            pltpu.VMEM((CHUNKS_PER_TILE, CHUNK), jnp.float32),
            pltpu.VMEM((NBUF, CHUNK, DH), jnp.float32),
            pltpu.VMEM((OUT_CHUNK, DH), jnp.float32),
            pltpu.VMEM_SHARED((N_NODES, DH), jnp.float32),
            pltpu.SemaphoreType.DMA((NBUF,)),
            pltpu.SemaphoreType.DMA((NBUF,)),
        ],
    )(tbl, w3, gi3, si3)
    return out.reshape(N_NODES, D_FEAT)
